# Initial kernel scaffold; baseline (speedup 1.0000x reference)
import jax
import jax.numpy as jnp
from jax.experimental import pallas as pl


def kernel(x, w, k_param):
    raise NotImplementedError("write your pallas kernel here")


# speedup vs baseline: 3.7982x; 3.7982x over previous
import jax
import jax.numpy as jnp
from jax.experimental import pallas as pl


def _stats_kernel(w_ref, k_ref, out_ref):
    wv = w_ref[...]
    m = jnp.max(wv)
    d = jnp.sum(jnp.exp(wv - m))
    coeff = jnp.round(k_ref[0, 0]) / (jnp.float32(wv.size) * d)
    out_ref[...] = jnp.stack([m, coeff]).reshape(1, 2)


def _wsum_kernel(stats_ref, w_ref, x_ref, out_ref):
    i = pl.program_id(0)
    m = stats_ref[0, 0]
    coeff = stats_ref[0, 1]
    e = jnp.exp(w_ref[0] - m) * coeff
    part = jax.lax.dot_general(
        e, x_ref[...], (((1,), (0,)), ((), ())),
        preferred_element_type=jnp.float32)

    @pl.when(i == 0)
    def _init():
        out_ref[...] = jnp.zeros_like(out_ref)

    out_ref[...] += part


def _pick_block(n):
    for b in (10000, 8000, 5000, 4096, 4000, 2048, 2000, 1000):
        if n % b == 0:
            return b
    return n


def kernel(x, w, k_param):
    n, t = x.shape
    b = _pick_block(n)
    g = n // b
    rows = 1000 if n % 1000 == 0 else 1

    stats = pl.pallas_call(
        _stats_kernel,
        out_shape=jax.ShapeDtypeStruct((1, 2), jnp.float32),
        in_specs=[
            pl.BlockSpec((n // rows, rows), lambda: (0, 0)),
            pl.BlockSpec((1, 1), lambda: (0, 0)),
        ],
        out_specs=pl.BlockSpec((1, 2), lambda: (0, 0)),
    )(w.reshape(n // rows, rows), k_param.reshape(1, 1))

    out = pl.pallas_call(
        _wsum_kernel,
        grid=(g,),
        out_shape=jax.ShapeDtypeStruct((1, t), jnp.float32),
        in_specs=[
            pl.BlockSpec((1, 2), lambda i: (0, 0)),
            pl.BlockSpec((1, 1, b), lambda i: (i, 0, 0)),
            pl.BlockSpec((b, t), lambda i: (i, 0)),
        ],
        out_specs=pl.BlockSpec((1, t), lambda i: (0, 0)),
    )(stats, w.reshape(g, 1, b), x)

    return out.reshape(t)


# speedup vs baseline: 3.8473x; 1.0129x over previous
import jax
import jax.numpy as jnp
from jax.experimental import pallas as pl


def _stats_kernel(w_ref, k_ref, out_ref):
    wv = w_ref[...]
    m = jnp.max(wv)
    d = jnp.sum(jnp.exp(wv - m))
    coeff = jnp.round(k_ref[0, 0]) / (jnp.float32(wv.size) * d)
    out_ref[...] = jnp.stack([m, coeff]).reshape(1, 2)


def _wsum_kernel(stats_ref, w_ref, x_ref, out_ref):
    i = pl.program_id(0)
    m = stats_ref[0, 0]
    coeff = stats_ref[0, 1]
    e = jnp.exp(w_ref[0] - m) * coeff
    part = jax.lax.dot_general(
        e, x_ref[...], (((1,), (0,)), ((), ())),
        preferred_element_type=jnp.float32)

    @pl.when(i == 0)
    def _init():
        out_ref[...] = jnp.zeros_like(out_ref)

    out_ref[...] += part


def _pick_block(n):
    for b in (40000, 10000, 8000, 5000, 4096, 4000, 2048, 2000, 1000):
        if n % b == 0:
            return b
    return n


def kernel(x, w, k_param):
    n, t = x.shape
    b = _pick_block(n)
    g = n // b
    rows = 1000 if n % 1000 == 0 else 1

    stats = pl.pallas_call(
        _stats_kernel,
        out_shape=jax.ShapeDtypeStruct((1, 2), jnp.float32),
        in_specs=[
            pl.BlockSpec((n // rows, rows), lambda: (0, 0)),
            pl.BlockSpec((1, 1), lambda: (0, 0)),
        ],
        out_specs=pl.BlockSpec((1, 2), lambda: (0, 0)),
    )(w.reshape(n // rows, rows), k_param.reshape(1, 1))

    out = pl.pallas_call(
        _wsum_kernel,
        grid=(g,),
        out_shape=jax.ShapeDtypeStruct((1, t), jnp.float32),
        in_specs=[
            pl.BlockSpec((1, 2), lambda i: (0, 0)),
            pl.BlockSpec((1, 1, b), lambda i: (i, 0, 0)),
            pl.BlockSpec((b, t), lambda i: (i, 0)),
        ],
        out_specs=pl.BlockSpec((1, t), lambda i: (0, 0)),
    )(stats, w.reshape(g, 1, b), x)

    return out.reshape(t)
